# baseline (speedup 1.0000x reference)
import functools

import jax
import jax.numpy as jnp
from jax import lax
from jax.experimental import pallas as pl
from jax.experimental.pallas import tpu as pltpu
from jax.experimental.pallas import tpu_sc as plsc

NUM_CORES = 2
NUM_SUBCORES = 16
LANES = 16
NW = NUM_CORES * NUM_SUBCORES

BATCH = 16384
EMBED_DIM = 64
B_PER_W = BATCH // NW

NUM_USERS = 1000000
NUM_MOVIES = 100000
CHUNK = 512
SHIFT = 9
NFULL_U = NUM_USERS // CHUNK
TAIL_U = NUM_USERS - NFULL_U * CHUNK
NFULL_M = NUM_MOVIES // CHUNK
TAIL_M = NUM_MOVIES - NFULL_M * CHUNK
TAIL_U_OWNER = NFULL_U % NW
TAIL_M_OWNER = NFULL_M % NW

RING = 32
NCHAIN = 4
QSTRIPS = BATCH // LANES // NCHAIN
QCAP = BATCH // NCHAIN + LANES
ROW_PAD = 128
NSTRIPS = BATCH // LANES


def _nr_rsqrt(p):
    i = lax.bitcast_convert_type(p, jnp.int32)
    i = jnp.int32(0x5F3759DF) - (i >> 1)
    y = lax.bitcast_convert_type(i, jnp.float32)
    for _ in range(3):
        y = y * (jnp.float32(1.5) - jnp.float32(0.5) * p * y * y)
    return y


def _harvest_body(utT, mtT, ui, mi, u_out, m_out,
                  idx_v, cap_p, sb_loc, sb_pos, buf_a, buf_b, hb64, hb32,
                  ring, sem, sem_a, sem_b):
    c = lax.axis_index("c")
    s = lax.axis_index("s")
    wid = s * NUM_CORES + c
    lane = lax.iota(jnp.int32, LANES)

    def retire(gw):
        @pl.when(gw >= LANES)
        def _():
            pltpu.make_async_copy(u_out.at[pl.ds(0, EMBED_DIM)],
                                  ring.at[0], sem).wait()

    def run_table(tab, out_ref, n_chunks, tail_owner, tail_subchunks):
        nfull = tab.shape[1] // CHUNK

        def capture(t, cs):
            out = []
            for q in range(NCHAIN):
                ts = t + q * QSTRIPS
                v = idx_v[pl.ds(ts * LANES, LANES)]
                mask = ((v >> SHIFT) & (NW - 1)) == wid
                plsc.store_compressed(
                    cap_p.at[pl.ds(q * QCAP + cs[q], LANES)],
                    ts * LANES + lane, mask=mask)
                out.append(cs[q] + plsc.all_reduce_population_count(mask)[0])
            return tuple(out)

        cnts = lax.fori_loop(0, QSTRIPS, capture,
                             (jnp.int32(0),) * NCHAIN)
        nstrips = [(cq + LANES - 1) // LANES for cq in cnts]

        def make_extract(buf):
            def extract_match(j, gw):
                u_loc = sb_loc[pl.ds(j, LANES)][0]
                pos = sb_pos[pl.ds(j, LANES)][0]
                retire(gw)
                slot = gw & (RING - 1)
                col = jnp.full((LANES,), 0, jnp.int32) + u_loc
                for q in range(EMBED_DIM // LANES):
                    vals = plsc.load_gather(buf, [lane + q * LANES, col])
                    ring[slot, pl.ds(q * LANES, LANES)] = vals
                pltpu.async_copy(
                    ring.at[slot],
                    out_ref.at[pl.ds(pos * ROW_PAD, EMBED_DIM)], sem)
                return gw + 1
            return extract_match

        def scan_chunk(k, off, width, buf, gw):
            extract = make_extract(buf)

            for q in range(NCHAIN):
                def strip(t, gw, q=q):
                    p = cap_p[pl.ds(q * QCAP + t * LANES, LANES)]
                    valid = (t * LANES + lane) < cnts[q]
                    v = plsc.load_gather(idx_v, [p], mask=valid)
                    loc = (v & (CHUNK - 1)) - off
                    mask = (valid & ((v >> SHIFT) == k)
                            & (loc >= 0) & (loc < width))
                    plsc.store_compressed(sb_loc.at[pl.ds(0, LANES)], loc,
                                          mask=mask)
                    plsc.store_compressed(sb_pos.at[pl.ds(0, LANES)], p,
                                          mask=mask)
                    m16 = plsc.all_reduce_population_count(mask)[0]
                    return lax.fori_loop(0, m16, extract, gw)

                gw = lax.fori_loop(0, nstrips[q], strip, gw)
            return gw

        def start_chunk(kk, buf, bsem):
            def go(_, carry):
                k = wid + NW * kk
                pltpu.async_copy(tab.at[:, pl.ds(k * CHUNK, CHUNK)],
                                 buf, bsem)
                return carry
            lax.fori_loop(0, (kk < n_chunks).astype(jnp.int32), go, 0)

        def wait_chunk(kk, buf, bsem):
            def go(_, carry):
                pltpu.make_async_copy(tab.at[:, pl.ds(0, CHUNK)],
                                      buf, bsem).wait()
                return carry
            lax.fori_loop(0, (kk < n_chunks).astype(jnp.int32), go, 0)

        def scan_if(kk, buf, gw):
            def go(_, gw):
                return scan_chunk(wid + NW * kk, 0, CHUNK, buf, gw)
            return lax.fori_loop(0, (kk < n_chunks).astype(jnp.int32),
                                 go, gw)

        start_chunk(jnp.int32(0), buf_a, sem_a)

        def pair(gg, gw):
            ka = 2 * gg
            wait_chunk(ka, buf_a, sem_a)
            start_chunk(ka + 1, buf_b, sem_b)
            gw = scan_if(ka, buf_a, gw)
            wait_chunk(ka + 1, buf_b, sem_b)
            start_chunk(ka + 2, buf_a, sem_a)
            gw = scan_if(ka + 1, buf_b, gw)
            return gw

        npair = (n_chunks + 1) // 2
        gw = lax.fori_loop(0, npair, pair, jnp.int32(0))

        do_tail = (wid == tail_owner).astype(jnp.int32)
        for off, width, buf, buf_is_slice in tail_subchunks:
            def tail_iter(_, gw, off=off, width=width, buf=buf,
                          buf_is_slice=buf_is_slice):
                dst = buf.at[:, pl.ds(0, width)] if buf_is_slice else buf
                pltpu.sync_copy(
                    tab.at[:, pl.ds(nfull * CHUNK + off, width)], dst)
                return scan_chunk(jnp.int32(nfull), off, width, buf, gw)

            gw = lax.fori_loop(0, do_tail, tail_iter, gw)

        def drain(_, g):
            pltpu.make_async_copy(u_out.at[pl.ds(0, EMBED_DIM)],
                                  ring.at[0], sem).wait()
            return g

        lax.fori_loop(0, jnp.minimum(gw, jnp.int32(LANES)), drain,
                      jnp.int32(0))
        return cnts[0]

    pltpu.sync_copy(ui, idx_v)
    run_table(utT, u_out, (NFULL_U - 1 - wid) // NW + 1, TAIL_U_OWNER,
              [(0, TAIL_U, hb64, False)])
    pltpu.sync_copy(mi, idx_v)
    run_table(mtT, m_out, (NFULL_M - 1 - wid) // NW + 1, TAIL_M_OWNER,
              [(0, 128, buf_a, True), (128, 32, hb32, False)])


def _cosine_body(u_ref, m_ref, o_ref):
    u = u_ref[:, :EMBED_DIM]
    m = m_ref[:, :EMBED_DIM]
    um = jnp.sum(u * m, axis=1)
    uu = jnp.sum(u * u, axis=1)
    mm = jnp.sum(m * m, axis=1)
    denom = (jnp.maximum(jnp.sqrt(uu), jnp.float32(1e-8))
             * jnp.maximum(jnp.sqrt(mm), jnp.float32(1e-8)))
    o_ref[...] = um / denom * jnp.float32(2.5) + jnp.float32(2.75)


def kernel(user_table, movie_table, user_idx, movie_idx):
    ui = user_idx.astype(jnp.int32)
    mi = movie_idx.astype(jnp.int32)
    mesh = plsc.VectorSubcoreMesh(core_axis_name="c", subcore_axis_name="s",
                                  num_cores=NUM_CORES,
                                  num_subcores=NUM_SUBCORES)
    params = pltpu.CompilerParams(needs_layout_passes=False)

    harvest = pl.kernel(
        _harvest_body,
        out_type=(jax.ShapeDtypeStruct((BATCH * ROW_PAD,), jnp.float32),
                  jax.ShapeDtypeStruct((BATCH * ROW_PAD,), jnp.float32)),
        mesh=mesh,
        compiler_params=params,
        scratch_types=[
            pltpu.VMEM((BATCH,), jnp.int32),
            pltpu.VMEM((NCHAIN * (BATCH // NCHAIN + LANES),), jnp.int32),
            pltpu.VMEM((2 * LANES,), jnp.int32),
            pltpu.VMEM((2 * LANES,), jnp.int32),
            pltpu.VMEM((EMBED_DIM, CHUNK), jnp.float32),
            pltpu.VMEM((EMBED_DIM, CHUNK), jnp.float32),
            pltpu.VMEM((EMBED_DIM, TAIL_U), jnp.float32),
            pltpu.VMEM((EMBED_DIM, 32), jnp.float32),
            pltpu.VMEM((RING, EMBED_DIM), jnp.float32),
            pltpu.SemaphoreType.DMA,
            pltpu.SemaphoreType.DMA,
            pltpu.SemaphoreType.DMA,
        ],
    )
    u_flat, m_flat = harvest(user_table.T, movie_table.T, ui, mi)

    COS_BLK = 2048
    cosine = pl.pallas_call(
        _cosine_body,
        grid=(BATCH // COS_BLK,),
        in_specs=[
            pl.BlockSpec((COS_BLK, ROW_PAD), lambda i: (i, 0)),
            pl.BlockSpec((COS_BLK, ROW_PAD), lambda i: (i, 0)),
        ],
        out_specs=pl.BlockSpec((COS_BLK,), lambda i: (i,)),
        out_shape=jax.ShapeDtypeStruct((BATCH,), jnp.float32),
    )
    return cosine(u_flat.reshape(BATCH, ROW_PAD),
                  m_flat.reshape(BATCH, ROW_PAD))


# speedup vs baseline: 1.0019x; 1.0019x over previous
import jax
import jax.numpy as jnp
from jax import lax
from jax.experimental import pallas as pl
from jax.experimental.pallas import tpu as pltpu
from jax.experimental.pallas import tpu_sc as plsc

NUM_CORES = 2
NUM_SUBCORES = 16
LANES = 16
NW = NUM_CORES * NUM_SUBCORES

BATCH = 16384
EMBED_DIM = 64
B_PER_W = BATCH // NW

NUM_USERS = 1000000
NUM_MOVIES = 100000
CHUNK = 512
SHIFT = 9
NFULL_U = NUM_USERS // CHUNK
TAIL_U = NUM_USERS - NFULL_U * CHUNK
NFULL_M = NUM_MOVIES // CHUNK
TAIL_M = NUM_MOVIES - NFULL_M * CHUNK
TAIL_U_OWNER = NFULL_U % NW
TAIL_M_OWNER = NFULL_M % NW

RING = 32
NCHAIN = 4
QSTRIPS = BATCH // LANES // NCHAIN
QCAP = BATCH // NCHAIN + LANES
ROW_PAD = 128


def _harvest_body(utT, mtT, ui, mi, u_out, m_out,
                  idx_v, cap_p, sb_loc, sb_pos, buf_a, buf_b, hb64, hb32,
                  ring, sem, sem_a, sem_b):
    c = lax.axis_index("c")
    s = lax.axis_index("s")
    wid = s * NUM_CORES + c
    lane = lax.iota(jnp.int32, LANES)

    def retire(gw):
        @pl.when(gw >= LANES)
        def _():
            pltpu.make_async_copy(u_out.at[pl.ds(0, EMBED_DIM)],
                                  ring.at[0], sem).wait()

    def run_table(tab, out_ref, n_chunks, tail_owner, tail_subchunks):
        nfull = tab.shape[1] // CHUNK

        def capture(t, cs):
            out = []
            for q in range(NCHAIN):
                ts = t + q * QSTRIPS
                v = idx_v[pl.ds(ts * LANES, LANES)]
                mask = ((v >> SHIFT) & (NW - 1)) == wid
                plsc.store_compressed(
                    cap_p.at[pl.ds(q * QCAP + cs[q], LANES)],
                    ts * LANES + lane, mask=mask)
                out.append(cs[q] + plsc.all_reduce_population_count(mask)[0])
            return tuple(out)

        cnts = lax.fori_loop(0, QSTRIPS, capture,
                             (jnp.int32(0),) * NCHAIN)
        nstrips = [(cq + LANES - 1) // LANES for cq in cnts]

        def make_extract(buf):
            def extract_match(j, gw):
                u_loc = sb_loc[pl.ds(j, LANES)][0]
                pos = sb_pos[pl.ds(j, LANES)][0]
                retire(gw)
                slot = gw & (RING - 1)
                col = jnp.full((LANES,), 0, jnp.int32) + u_loc
                for q in range(EMBED_DIM // LANES):
                    vals = plsc.load_gather(buf, [lane + q * LANES, col])
                    ring[slot, pl.ds(q * LANES, LANES)] = vals
                pltpu.async_copy(
                    ring.at[slot],
                    out_ref.at[pl.ds(pos * ROW_PAD, EMBED_DIM)], sem)
                return gw + 1
            return extract_match

        def scan_chunk(k, off, width, buf, gw):
            extract = make_extract(buf)

            for q in range(NCHAIN):
                def strip(t, gw, q=q):
                    p = cap_p[pl.ds(q * QCAP + t * LANES, LANES)]
                    valid = (t * LANES + lane) < cnts[q]
                    v = plsc.load_gather(idx_v, [p], mask=valid)
                    loc = (v & (CHUNK - 1)) - off
                    mask = (valid & ((v >> SHIFT) == k)
                            & (loc >= 0) & (loc < width))
                    plsc.store_compressed(sb_loc.at[pl.ds(0, LANES)], loc,
                                          mask=mask)
                    plsc.store_compressed(sb_pos.at[pl.ds(0, LANES)], p,
                                          mask=mask)
                    m16 = plsc.all_reduce_population_count(mask)[0]
                    return lax.fori_loop(0, m16, extract, gw)

                gw = lax.fori_loop(0, nstrips[q], strip, gw)
            return gw

        def start_chunk(kk, buf, bsem):
            def go(_, carry):
                k = wid + NW * kk
                pltpu.async_copy(tab.at[:, pl.ds(k * CHUNK, CHUNK)],
                                 buf, bsem)
                return carry
            lax.fori_loop(0, (kk < n_chunks).astype(jnp.int32), go, 0)

        def wait_chunk(kk, buf, bsem):
            def go(_, carry):
                pltpu.make_async_copy(tab.at[:, pl.ds(0, CHUNK)],
                                      buf, bsem).wait()
                return carry
            lax.fori_loop(0, (kk < n_chunks).astype(jnp.int32), go, 0)

        def scan_if(kk, buf, gw):
            def go(_, gw):
                return scan_chunk(wid + NW * kk, 0, CHUNK, buf, gw)
            return lax.fori_loop(0, (kk < n_chunks).astype(jnp.int32),
                                 go, gw)

        start_chunk(jnp.int32(0), buf_a, sem_a)

        def pair(gg, gw):
            ka = 2 * gg
            wait_chunk(ka, buf_a, sem_a)
            start_chunk(ka + 1, buf_b, sem_b)
            gw = scan_if(ka, buf_a, gw)
            wait_chunk(ka + 1, buf_b, sem_b)
            start_chunk(ka + 2, buf_a, sem_a)
            gw = scan_if(ka + 1, buf_b, gw)
            return gw

        npair = (n_chunks + 1) // 2
        gw = lax.fori_loop(0, npair, pair, jnp.int32(0))

        do_tail = (wid == tail_owner).astype(jnp.int32)
        for off, width, buf, buf_is_slice in tail_subchunks:
            def tail_iter(_, gw, off=off, width=width, buf=buf,
                          buf_is_slice=buf_is_slice):
                dst = buf.at[:, pl.ds(0, width)] if buf_is_slice else buf
                pltpu.sync_copy(
                    tab.at[:, pl.ds(nfull * CHUNK + off, width)], dst)
                return scan_chunk(jnp.int32(nfull), off, width, buf, gw)

            gw = lax.fori_loop(0, do_tail, tail_iter, gw)

        def drain(_, g):
            pltpu.make_async_copy(u_out.at[pl.ds(0, EMBED_DIM)],
                                  ring.at[0], sem).wait()
            return g

        lax.fori_loop(0, jnp.minimum(gw, jnp.int32(LANES)), drain,
                      jnp.int32(0))
        return cnts[0]

    pltpu.sync_copy(ui, idx_v)
    run_table(utT, u_out, (NFULL_U - 1 - wid) // NW + 1, TAIL_U_OWNER,
              [(0, TAIL_U, hb64, False)])
    pltpu.sync_copy(mi, idx_v)
    run_table(mtT, m_out, (NFULL_M - 1 - wid) // NW + 1, TAIL_M_OWNER,
              [(0, 128, buf_a, True), (128, 32, hb32, False)])


def _cosine_body(u_ref, m_ref, o_ref):
    u = u_ref[:, :EMBED_DIM]
    m = m_ref[:, :EMBED_DIM]
    um = jnp.sum(u * m, axis=1)
    uu = jnp.sum(u * u, axis=1)
    mm = jnp.sum(m * m, axis=1)
    denom = (jnp.maximum(jnp.sqrt(uu), jnp.float32(1e-8))
             * jnp.maximum(jnp.sqrt(mm), jnp.float32(1e-8)))
    o_ref[...] = um / denom * jnp.float32(2.5) + jnp.float32(2.75)


def kernel(user_table, movie_table, user_idx, movie_idx):
    ui = user_idx.astype(jnp.int32)
    mi = movie_idx.astype(jnp.int32)
    mesh = plsc.VectorSubcoreMesh(core_axis_name="c", subcore_axis_name="s",
                                  num_cores=NUM_CORES,
                                  num_subcores=NUM_SUBCORES)
    params = pltpu.CompilerParams(needs_layout_passes=False)

    harvest = pl.kernel(
        _harvest_body,
        out_type=(jax.ShapeDtypeStruct((BATCH * ROW_PAD,), jnp.float32),
                  jax.ShapeDtypeStruct((BATCH * ROW_PAD,), jnp.float32)),
        mesh=mesh,
        compiler_params=params,
        scratch_types=[
            pltpu.VMEM((BATCH,), jnp.int32),
            pltpu.VMEM((NCHAIN * (BATCH // NCHAIN + LANES),), jnp.int32),
            pltpu.VMEM((2 * LANES,), jnp.int32),
            pltpu.VMEM((2 * LANES,), jnp.int32),
            pltpu.VMEM((EMBED_DIM, CHUNK), jnp.float32),
            pltpu.VMEM((EMBED_DIM, CHUNK), jnp.float32),
            pltpu.VMEM((EMBED_DIM, TAIL_U), jnp.float32),
            pltpu.VMEM((EMBED_DIM, 32), jnp.float32),
            pltpu.VMEM((RING, EMBED_DIM), jnp.float32),
            pltpu.SemaphoreType.DMA,
            pltpu.SemaphoreType.DMA,
            pltpu.SemaphoreType.DMA,
        ],
    )
    u_flat, m_flat = harvest(user_table.T, movie_table.T, ui, mi)

    COS_BLK = 2048
    cosine = pl.pallas_call(
        _cosine_body,
        grid=(BATCH // COS_BLK,),
        in_specs=[
            pl.BlockSpec((COS_BLK, ROW_PAD), lambda i: (i, 0)),
            pl.BlockSpec((COS_BLK, ROW_PAD), lambda i: (i, 0)),
        ],
        out_specs=pl.BlockSpec((COS_BLK,), lambda i: (i,)),
        out_shape=jax.ShapeDtypeStruct((BATCH,), jnp.float32),
    )
    return cosine(u_flat.reshape(BATCH, ROW_PAD),
                  m_flat.reshape(BATCH, ROW_PAD))


# speedup vs baseline: 1.0257x; 1.0238x over previous
import jax
import jax.numpy as jnp
from jax import lax
from jax.experimental import pallas as pl
from jax.experimental.pallas import tpu as pltpu
from jax.experimental.pallas import tpu_sc as plsc

NUM_CORES = 2
NUM_SUBCORES = 16
LANES = 16
NW = NUM_CORES * NUM_SUBCORES

BATCH = 16384
EMBED_DIM = 64
B_PER_W = BATCH // NW

NUM_USERS = 1000000
NUM_MOVIES = 100000
CHUNK = 512
SHIFT = 9
NFULL_U = NUM_USERS // CHUNK
TAIL_U = NUM_USERS - NFULL_U * CHUNK
NFULL_M = NUM_MOVIES // CHUNK
TAIL_M = NUM_MOVIES - NFULL_M * CHUNK
TAIL_U_OWNER = NFULL_U % NW
TAIL_M_OWNER = NFULL_M % NW

RING = 32
NCHAIN = 4
QSTRIPS = BATCH // LANES // NCHAIN
QCAP = BATCH // NCHAIN + LANES
ROW_PAD = 128


def _harvest_body(utT, mtT, ui, mi, u_out, m_out,
                  idx_v, cap_p, sb_loc, sb_pos, buf_a, buf_b, hb64, hb32,
                  ring, sem, sem_a, sem_b):
    c = lax.axis_index("c")
    s = lax.axis_index("s")
    wid = s * NUM_CORES + c
    lane = lax.iota(jnp.int32, LANES)

    def retire(gw):
        @pl.when(gw >= LANES)
        def _():
            pltpu.make_async_copy(u_out.at[pl.ds(0, EMBED_DIM)],
                                  ring.at[0], sem).wait()

    def run_table(tab, out_ref, n_chunks, tail_owner, tail_subchunks):
        nfull = tab.shape[1] // CHUNK

        def capture(t, cs):
            out = []
            for q in range(NCHAIN):
                ts = t + q * QSTRIPS
                v = idx_v[pl.ds(ts * LANES, LANES)]
                mask = ((v >> SHIFT) & (NW - 1)) == wid
                plsc.store_compressed(
                    cap_p.at[pl.ds(q * QCAP + cs[q], LANES)],
                    ts * LANES + lane, mask=mask)
                out.append(cs[q] + plsc.all_reduce_population_count(mask)[0])
            return tuple(out)

        cnts = lax.fori_loop(0, QSTRIPS, capture,
                             (jnp.int32(0),) * NCHAIN)
        nstrips = [(cq + LANES - 1) // LANES for cq in cnts]

        def make_extract(buf):
            def extract_match(j, gw):
                u_loc = sb_loc[pl.ds(j, LANES)][0]
                pos = sb_pos[pl.ds(j, LANES)][0]
                retire(gw)
                slot = gw & (RING - 1)
                col = jnp.full((LANES,), 0, jnp.int32) + u_loc
                for q in range(EMBED_DIM // LANES):
                    vals = plsc.load_gather(buf, [lane + q * LANES, col])
                    ring[slot, pl.ds(q * LANES, LANES)] = vals
                pltpu.async_copy(
                    ring.at[slot],
                    out_ref.at[pl.ds(pos * ROW_PAD, EMBED_DIM)], sem)
                return gw + 1
            return extract_match

        def scan_chunk(k, off, width, buf, gw):
            extract = make_extract(buf)

            def strip(t, gw):
                hits = []
                for q in range(NCHAIN):
                    p = cap_p[pl.ds(q * QCAP + t * LANES, LANES)]
                    valid = (t * LANES + lane) < cnts[q]
                    v = plsc.load_gather(idx_v, [p], mask=valid)
                    loc = (v & (CHUNK - 1)) - off
                    mask = (valid & ((v >> SHIFT) == k)
                            & (loc >= 0) & (loc < width))
                    m16 = plsc.all_reduce_population_count(mask)[0]
                    hits.append((p, loc, mask, m16))
                for p, loc, mask, m16 in hits:
                    plsc.store_compressed(sb_loc.at[pl.ds(0, LANES)], loc,
                                          mask=mask)
                    plsc.store_compressed(sb_pos.at[pl.ds(0, LANES)], p,
                                          mask=mask)
                    gw = lax.fori_loop(0, m16, extract, gw)
                return gw

            nstrip = jnp.maximum(jnp.maximum(nstrips[0], nstrips[1]),
                                 jnp.maximum(nstrips[2], nstrips[3]))
            return lax.fori_loop(0, nstrip, strip, gw)

        def start_chunk(kk, buf, bsem):
            def go(_, carry):
                k = wid + NW * kk
                pltpu.async_copy(tab.at[:, pl.ds(k * CHUNK, CHUNK)],
                                 buf, bsem)
                return carry
            lax.fori_loop(0, (kk < n_chunks).astype(jnp.int32), go, 0)

        def wait_chunk(kk, buf, bsem):
            def go(_, carry):
                pltpu.make_async_copy(tab.at[:, pl.ds(0, CHUNK)],
                                      buf, bsem).wait()
                return carry
            lax.fori_loop(0, (kk < n_chunks).astype(jnp.int32), go, 0)

        def scan_if(kk, buf, gw):
            def go(_, gw):
                return scan_chunk(wid + NW * kk, 0, CHUNK, buf, gw)
            return lax.fori_loop(0, (kk < n_chunks).astype(jnp.int32),
                                 go, gw)

        start_chunk(jnp.int32(0), buf_a, sem_a)

        def pair(gg, gw):
            ka = 2 * gg
            wait_chunk(ka, buf_a, sem_a)
            start_chunk(ka + 1, buf_b, sem_b)
            gw = scan_if(ka, buf_a, gw)
            wait_chunk(ka + 1, buf_b, sem_b)
            start_chunk(ka + 2, buf_a, sem_a)
            gw = scan_if(ka + 1, buf_b, gw)
            return gw

        npair = (n_chunks + 1) // 2
        gw = lax.fori_loop(0, npair, pair, jnp.int32(0))

        do_tail = (wid == tail_owner).astype(jnp.int32)
        for off, width, buf, buf_is_slice in tail_subchunks:
            def tail_iter(_, gw, off=off, width=width, buf=buf,
                          buf_is_slice=buf_is_slice):
                dst = buf.at[:, pl.ds(0, width)] if buf_is_slice else buf
                pltpu.sync_copy(
                    tab.at[:, pl.ds(nfull * CHUNK + off, width)], dst)
                return scan_chunk(jnp.int32(nfull), off, width, buf, gw)

            gw = lax.fori_loop(0, do_tail, tail_iter, gw)

        def drain(_, g):
            pltpu.make_async_copy(u_out.at[pl.ds(0, EMBED_DIM)],
                                  ring.at[0], sem).wait()
            return g

        lax.fori_loop(0, jnp.minimum(gw, jnp.int32(LANES)), drain,
                      jnp.int32(0))
        return cnts[0]

    pltpu.sync_copy(ui, idx_v)
    run_table(utT, u_out, (NFULL_U - 1 - wid) // NW + 1, TAIL_U_OWNER,
              [(0, TAIL_U, hb64, False)])
    pltpu.sync_copy(mi, idx_v)
    run_table(mtT, m_out, (NFULL_M - 1 - wid) // NW + 1, TAIL_M_OWNER,
              [(0, 128, buf_a, True), (128, 32, hb32, False)])


def _cosine_body(u_ref, m_ref, o_ref):
    u = u_ref[:, :EMBED_DIM]
    m = m_ref[:, :EMBED_DIM]
    um = jnp.sum(u * m, axis=1)
    uu = jnp.sum(u * u, axis=1)
    mm = jnp.sum(m * m, axis=1)
    denom = (jnp.maximum(jnp.sqrt(uu), jnp.float32(1e-8))
             * jnp.maximum(jnp.sqrt(mm), jnp.float32(1e-8)))
    o_ref[...] = um / denom * jnp.float32(2.5) + jnp.float32(2.75)


def kernel(user_table, movie_table, user_idx, movie_idx):
    ui = user_idx.astype(jnp.int32)
    mi = movie_idx.astype(jnp.int32)
    mesh = plsc.VectorSubcoreMesh(core_axis_name="c", subcore_axis_name="s",
                                  num_cores=NUM_CORES,
                                  num_subcores=NUM_SUBCORES)
    params = pltpu.CompilerParams(needs_layout_passes=False)

    harvest = pl.kernel(
        _harvest_body,
        out_type=(jax.ShapeDtypeStruct((BATCH * ROW_PAD,), jnp.float32),
                  jax.ShapeDtypeStruct((BATCH * ROW_PAD,), jnp.float32)),
        mesh=mesh,
        compiler_params=params,
        scratch_types=[
            pltpu.VMEM((BATCH,), jnp.int32),
            pltpu.VMEM((NCHAIN * (BATCH // NCHAIN + LANES),), jnp.int32),
            pltpu.VMEM((2 * LANES,), jnp.int32),
            pltpu.VMEM((2 * LANES,), jnp.int32),
            pltpu.VMEM((EMBED_DIM, CHUNK), jnp.float32),
            pltpu.VMEM((EMBED_DIM, CHUNK), jnp.float32),
            pltpu.VMEM((EMBED_DIM, TAIL_U), jnp.float32),
            pltpu.VMEM((EMBED_DIM, 32), jnp.float32),
            pltpu.VMEM((RING, EMBED_DIM), jnp.float32),
            pltpu.SemaphoreType.DMA,
            pltpu.SemaphoreType.DMA,
            pltpu.SemaphoreType.DMA,
        ],
    )
    u_flat, m_flat = harvest(user_table.T, movie_table.T, ui, mi)

    COS_BLK = 2048
    cosine = pl.pallas_call(
        _cosine_body,
        grid=(BATCH // COS_BLK,),
        in_specs=[
            pl.BlockSpec((COS_BLK, ROW_PAD), lambda i: (i, 0)),
            pl.BlockSpec((COS_BLK, ROW_PAD), lambda i: (i, 0)),
        ],
        out_specs=pl.BlockSpec((COS_BLK,), lambda i: (i,)),
        out_shape=jax.ShapeDtypeStruct((BATCH,), jnp.float32),
    )
    return cosine(u_flat.reshape(BATCH, ROW_PAD),
                  m_flat.reshape(BATCH, ROW_PAD))


# speedup vs baseline: 1.0280x; 1.0022x over previous
import jax
import jax.numpy as jnp
from jax import lax
from jax.experimental import pallas as pl
from jax.experimental.pallas import tpu as pltpu
from jax.experimental.pallas import tpu_sc as plsc

NUM_CORES = 2
NUM_SUBCORES = 16
LANES = 16
NW = NUM_CORES * NUM_SUBCORES

BATCH = 16384
EMBED_DIM = 64
B_PER_W = BATCH // NW

NUM_USERS = 1000000
NUM_MOVIES = 100000
CHUNK = 512
SHIFT = 9
NFULL_U = NUM_USERS // CHUNK
TAIL_U = NUM_USERS - NFULL_U * CHUNK
NFULL_M = NUM_MOVIES // CHUNK
TAIL_M = NUM_MOVIES - NFULL_M * CHUNK
TAIL_U_OWNER = NFULL_U % NW
TAIL_M_OWNER = NFULL_M % NW

RING = 32
NCHAIN = 4
QSTRIPS = BATCH // LANES // NCHAIN
QCAP = BATCH // NCHAIN + LANES
ROW_PAD = 128


def _harvest_body(utT, mtT, ui, mi, u_out, m_out,
                  idx_v, cap_p, sb_loc, sb_pos, buf_a, buf_b, hb64, hb32,
                  ring, sem, sem_a, sem_b):
    c = lax.axis_index("c")
    s = lax.axis_index("s")
    wid = s * NUM_CORES + c
    lane = lax.iota(jnp.int32, LANES)

    def retire(gw):
        @pl.when(gw >= LANES)
        def _():
            pltpu.make_async_copy(u_out.at[pl.ds(0, EMBED_DIM)],
                                  ring.at[0], sem).wait()

    def run_table(tab, out_ref, n_chunks, tail_owner, tail_subchunks):
        nfull = tab.shape[1] // CHUNK

        def capture(t, cs):
            out = []
            for q in range(NCHAIN):
                ts = t + q * QSTRIPS
                v = idx_v[pl.ds(ts * LANES, LANES)]
                mask = ((v >> SHIFT) & (NW - 1)) == wid
                plsc.store_compressed(
                    cap_p.at[pl.ds(q * QCAP + cs[q], LANES)],
                    ts * LANES + lane, mask=mask)
                out.append(cs[q] + plsc.all_reduce_population_count(mask)[0])
            return tuple(out)

        cnts = lax.fori_loop(0, QSTRIPS, capture,
                             (jnp.int32(0),) * NCHAIN)
        nstrips = [(cq + LANES - 1) // LANES for cq in cnts]

        def make_extract(buf):
            def extract_match(j, gw):
                u_loc = sb_loc[pl.ds(j, LANES)][0]
                pos = sb_pos[pl.ds(j, LANES)][0]
                retire(gw)
                slot = gw & (RING - 1)
                col = jnp.full((LANES,), 0, jnp.int32) + u_loc
                for q in range(EMBED_DIM // LANES):
                    vals = plsc.load_gather(buf, [lane + q * LANES, col])
                    ring[slot, pl.ds(q * LANES, LANES)] = vals
                pltpu.async_copy(
                    ring.at[slot],
                    out_ref.at[pl.ds(pos * ROW_PAD, EMBED_DIM)], sem)
                return gw + 1
            return extract_match

        def scan_chunk(k, off, width, buf, gw):
            extract = make_extract(buf)

            def strip(t, gw):
                hits = []
                for q in range(NCHAIN):
                    p = cap_p[pl.ds(q * QCAP + t * LANES, LANES)]
                    valid = (t * LANES + lane) < cnts[q]
                    v = plsc.load_gather(idx_v, [p], mask=valid)
                    loc = (v & (CHUNK - 1)) - off
                    mask = (valid & ((v >> SHIFT) == k)
                            & (loc >= 0) & (loc < width))
                    m16 = plsc.all_reduce_population_count(mask)[0]
                    hits.append((p, loc, mask, m16))
                for p, loc, mask, m16 in hits:
                    plsc.store_compressed(sb_loc.at[pl.ds(0, LANES)], loc,
                                          mask=mask)
                    plsc.store_compressed(sb_pos.at[pl.ds(0, LANES)], p,
                                          mask=mask)
                    gw = lax.fori_loop(0, m16, extract, gw)
                return gw

            nstrip = jnp.maximum(jnp.maximum(nstrips[0], nstrips[1]),
                                 jnp.maximum(nstrips[2], nstrips[3]))
            return lax.fori_loop(0, nstrip, strip, gw)

        def start_chunk(kk, buf, bsem):
            def go(_, carry):
                k = wid + NW * kk
                pltpu.async_copy(tab.at[:, pl.ds(k * CHUNK, CHUNK)],
                                 buf, bsem)
                return carry
            lax.fori_loop(0, (kk < n_chunks).astype(jnp.int32), go, 0)

        def wait_chunk(kk, buf, bsem):
            def go(_, carry):
                pltpu.make_async_copy(tab.at[:, pl.ds(0, CHUNK)],
                                      buf, bsem).wait()
                return carry
            lax.fori_loop(0, (kk < n_chunks).astype(jnp.int32), go, 0)

        def scan_if(kk, buf, gw):
            def go(_, gw):
                return scan_chunk(wid + NW * kk, 0, CHUNK, buf, gw)
            return lax.fori_loop(0, (kk < n_chunks).astype(jnp.int32),
                                 go, gw)

        start_chunk(jnp.int32(0), buf_a, sem_a)

        def pair(gg, gw):
            ka = 2 * gg
            wait_chunk(ka, buf_a, sem_a)
            start_chunk(ka + 1, buf_b, sem_b)
            gw = scan_if(ka, buf_a, gw)
            wait_chunk(ka + 1, buf_b, sem_b)
            start_chunk(ka + 2, buf_a, sem_a)
            gw = scan_if(ka + 1, buf_b, gw)
            return gw

        npair = (n_chunks + 1) // 2
        gw = lax.fori_loop(0, npair, pair, jnp.int32(0))

        do_tail = (wid == tail_owner).astype(jnp.int32)
        for off, width, buf, buf_is_slice in tail_subchunks:
            def tail_iter(_, gw, off=off, width=width, buf=buf,
                          buf_is_slice=buf_is_slice):
                dst = buf.at[:, pl.ds(0, width)] if buf_is_slice else buf
                pltpu.sync_copy(
                    tab.at[:, pl.ds(nfull * CHUNK + off, width)], dst)
                return scan_chunk(jnp.int32(nfull), off, width, buf, gw)

            gw = lax.fori_loop(0, do_tail, tail_iter, gw)

        def drain(_, g):
            pltpu.make_async_copy(u_out.at[pl.ds(0, EMBED_DIM)],
                                  ring.at[0], sem).wait()
            return g

        lax.fori_loop(0, jnp.minimum(gw, jnp.int32(LANES)), drain,
                      jnp.int32(0))
        return cnts[0]

    pltpu.sync_copy(ui, idx_v)
    run_table(utT, u_out, (NFULL_U - 1 - wid) // NW + 1, TAIL_U_OWNER,
              [(0, TAIL_U, hb64, False)])
    pltpu.sync_copy(mi, idx_v)
    run_table(mtT, m_out, (NFULL_M - 1 - wid) // NW + 1, TAIL_M_OWNER,
              [(0, 128, buf_a, True), (128, 32, hb32, False)])


def _cosine_body(u_ref, m_ref, o_ref):
    u = u_ref[:, :EMBED_DIM]
    m = m_ref[:, :EMBED_DIM]
    um = jnp.sum(u * m, axis=1)
    uu = jnp.sum(u * u, axis=1)
    mm = jnp.sum(m * m, axis=1)
    denom = (jnp.maximum(jnp.sqrt(uu), jnp.float32(1e-8))
             * jnp.maximum(jnp.sqrt(mm), jnp.float32(1e-8)))
    o_ref[...] = um / denom * jnp.float32(2.5) + jnp.float32(2.75)


def kernel(user_table, movie_table, user_idx, movie_idx):
    ui = user_idx.astype(jnp.int32)
    mi = movie_idx.astype(jnp.int32)
    mesh = plsc.VectorSubcoreMesh(core_axis_name="c", subcore_axis_name="s",
                                  num_cores=NUM_CORES,
                                  num_subcores=NUM_SUBCORES)
    params = pltpu.CompilerParams(needs_layout_passes=False)

    harvest = pl.kernel(
        _harvest_body,
        out_type=(jax.ShapeDtypeStruct((BATCH * ROW_PAD,), jnp.float32),
                  jax.ShapeDtypeStruct((BATCH * ROW_PAD,), jnp.float32)),
        mesh=mesh,
        compiler_params=params,
        scratch_types=[
            pltpu.VMEM((BATCH,), jnp.int32),
            pltpu.VMEM((NCHAIN * (BATCH // NCHAIN + LANES),), jnp.int32),
            pltpu.VMEM((2 * LANES,), jnp.int32),
            pltpu.VMEM((2 * LANES,), jnp.int32),
            pltpu.VMEM((EMBED_DIM, CHUNK), jnp.float32),
            pltpu.VMEM((EMBED_DIM, CHUNK), jnp.float32),
            pltpu.VMEM((EMBED_DIM, TAIL_U), jnp.float32),
            pltpu.VMEM((EMBED_DIM, 32), jnp.float32),
            pltpu.VMEM((RING, EMBED_DIM), jnp.float32),
            pltpu.SemaphoreType.DMA,
            pltpu.SemaphoreType.DMA,
            pltpu.SemaphoreType.DMA,
        ],
    )
    u_flat, m_flat = harvest(user_table.T, movie_table.T, ui, mi)

    COS_BLK = 4096
    cosine = pl.pallas_call(
        _cosine_body,
        grid=(BATCH // COS_BLK,),
        in_specs=[
            pl.BlockSpec((COS_BLK, ROW_PAD), lambda i: (i, 0)),
            pl.BlockSpec((COS_BLK, ROW_PAD), lambda i: (i, 0)),
        ],
        out_specs=pl.BlockSpec((COS_BLK,), lambda i: (i,)),
        out_shape=jax.ShapeDtypeStruct((BATCH,), jnp.float32),
    )
    return cosine(u_flat.reshape(BATCH, ROW_PAD),
                  m_flat.reshape(BATCH, ROW_PAD))
